# baseline (speedup 1.0000x reference)
import functools

import jax
import jax.numpy as jnp
from jax import lax
from jax.experimental import pallas as pl
from jax.experimental.pallas import tpu as pltpu
from jax.experimental.pallas import tpu_sc as plsc

_IDX_CHUNK = 128


def _make_sc_gather(batch, emb_dim, n_rows):
    info = plsc.get_sparse_core_info()
    nc, ns = info.num_cores, info.num_subcores
    nw = nc * ns
    b_per_w = batch // nw
    n_chunks = b_per_w // _IDX_CHUNK
    mesh = plsc.VectorSubcoreMesh(core_axis_name="c", subcore_axis_name="s")

    @functools.partial(
        pl.kernel,
        mesh=mesh,
        compiler_params=pltpu.CompilerParams(use_tc_tiling_on_sc=False),
        out_type=jax.ShapeDtypeStruct((batch, emb_dim), jnp.float32),
        scratch_types=[
            pltpu.VMEM((n_chunks, _IDX_CHUNK), jnp.int32),
            pltpu.VMEM((b_per_w, emb_dim), jnp.float32),
            pltpu.SemaphoreType.DMA,
        ],
    )
    def sc_gather(idx_hbm, table_hbm, out_hbm, idx_v, rows_v, sem):
        wid = lax.axis_index("s") * nc + lax.axis_index("c")
        base = wid * b_per_w
        pltpu.sync_copy(idx_hbm.at[wid], idx_v)
        copies = []
        for j in range(n_chunks):
            copies.append(
                pltpu.async_copy(
                    table_hbm.at[idx_v.at[j]],
                    rows_v.at[pl.ds(j * _IDX_CHUNK, _IDX_CHUNK)],
                    sem,
                )
            )
        for c in copies:
            c.wait()
        pltpu.sync_copy(rows_v, out_hbm.at[pl.ds(base, b_per_w)])

    return sc_gather, nw, b_per_w


def _tc_body(feats_ref, emb_ref, w_ref, b_ref, out_ref):
    y = lax.dot_general(
        emb_ref[...], w_ref[...],
        (((1,), (1,)), ((), ())),
        preferred_element_type=jnp.float32,
    )
    out_ref[:, : feats_ref.shape[1]] = feats_ref[...]
    out_ref[:, feats_ref.shape[1]:] = y + b_ref[...]


def kernel(ids, feats, layer_idx, emb_table, W, b):
    batch = ids.shape[0]
    input_dim = feats.shape[1]
    emb_dim = W.shape[0]
    n_nodes = emb_table.shape[0] - 1

    lookup = jnp.where(layer_idx > 0, ids, n_nodes).astype(jnp.int32)

    sc_gather, nw, b_per_w = _make_sc_gather(batch, emb_dim, emb_table.shape[0])
    idx3 = lookup.reshape(nw, b_per_w // _IDX_CHUNK, _IDX_CHUNK)
    gathered = sc_gather(idx3, emb_table)

    bb = 2048
    grid = (batch // bb,)
    out = pl.pallas_call(
        _tc_body,
        grid=grid,
        in_specs=[
            pl.BlockSpec((bb, input_dim), lambda i: (i, 0)),
            pl.BlockSpec((bb, emb_dim), lambda i: (i, 0)),
            pl.BlockSpec((emb_dim, emb_dim), lambda i: (0, 0)),
            pl.BlockSpec((1, emb_dim), lambda i: (0, 0)),
        ],
        out_specs=pl.BlockSpec((bb, input_dim + emb_dim), lambda i: (i, 0)),
        out_shape=jax.ShapeDtypeStruct((batch, input_dim + emb_dim), jnp.float32),
    )(feats, gathered, W, b.reshape(1, emb_dim))
    return out


# speedup vs baseline: 1.0208x; 1.0208x over previous
import functools

import jax
import jax.numpy as jnp
from jax import lax
from jax.experimental import pallas as pl
from jax.experimental.pallas import tpu as pltpu
from jax.experimental.pallas import tpu_sc as plsc

_IDX_CHUNK = 128


def _make_sc_gather(batch, emb_dim):
    info = plsc.get_sparse_core_info()
    nc, ns = info.num_cores, info.num_subcores
    nw = nc * ns
    b_per_w = batch // nw
    n_chunks = b_per_w // _IDX_CHUNK
    mesh = plsc.VectorSubcoreMesh(core_axis_name="c", subcore_axis_name="s")

    @functools.partial(
        pl.kernel,
        mesh=mesh,
        compiler_params=pltpu.CompilerParams(use_tc_tiling_on_sc=False),
        out_type=jax.ShapeDtypeStruct((batch, emb_dim), jnp.float32),
        scratch_types=[
            pltpu.VMEM((n_chunks, _IDX_CHUNK), jnp.int32),
            pltpu.VMEM((b_per_w, emb_dim), jnp.float32),
            pltpu.SemaphoreType.DMA,
        ],
    )
    def sc_gather(idx_hbm, table_hbm, out_hbm, idx_v, rows_v, sem):
        wid = lax.axis_index("s") * nc + lax.axis_index("c")
        base = wid * b_per_w
        pltpu.sync_copy(idx_hbm.at[wid], idx_v)
        copies = []
        for j in range(n_chunks):
            copies.append(
                pltpu.async_copy(
                    table_hbm.at[idx_v.at[j]],
                    rows_v.at[pl.ds(j * _IDX_CHUNK, _IDX_CHUNK)],
                    sem,
                )
            )
        for c in copies:
            c.wait()
        pltpu.sync_copy(rows_v, out_hbm.at[pl.ds(base, b_per_w)])

    return sc_gather, nw, b_per_w


def _tc_body(g_ref, f_ref, w_ref, b_ref, eye_ref, out_ref):
    input_dim = f_ref.shape[1]
    fT = lax.dot_general(
        eye_ref[...], f_ref[...],
        (((1,), (1,)), ((), ())),
        preferred_element_type=jnp.float32,
    )
    yT = lax.dot_general(
        w_ref[...], g_ref[...],
        (((1,), (1,)), ((), ())),
        preferred_element_type=jnp.float32,
    )
    out_ref[:input_dim, :] = fT
    out_ref[input_dim:, :] = yT + b_ref[...]


def kernel(ids, feats, layer_idx, emb_table, W, b):
    batch = ids.shape[0]
    input_dim = feats.shape[1]
    emb_dim = W.shape[0]
    n_nodes = emb_table.shape[0] - 1

    lookup = jnp.where(layer_idx > 0, ids, n_nodes).astype(jnp.int32)

    sc_gather, nw, b_per_w = _make_sc_gather(batch, emb_dim)
    idx3 = lookup.reshape(nw, b_per_w // _IDX_CHUNK, _IDX_CHUNK)
    gathered = sc_gather(idx3, emb_table)

    bb = 2048
    out_dim = input_dim + emb_dim
    eye = jnp.eye(input_dim, dtype=jnp.float32)
    outT = pl.pallas_call(
        _tc_body,
        grid=(batch // bb,),
        in_specs=[
            pl.BlockSpec((bb, emb_dim), lambda i: (i, 0)),
            pl.BlockSpec((bb, input_dim), lambda i: (i, 0)),
            pl.BlockSpec((emb_dim, emb_dim), lambda i: (0, 0)),
            pl.BlockSpec((emb_dim, 1), lambda i: (0, 0)),
            pl.BlockSpec((input_dim, input_dim), lambda i: (0, 0)),
        ],
        out_specs=pl.BlockSpec((out_dim, bb), lambda i: (0, i)),
        out_shape=jax.ShapeDtypeStruct((out_dim, batch), jnp.float32),
    )(gathered, feats, W, b.reshape(emb_dim, 1), eye)
    return outT.T
